# baseline (speedup 1.0000x reference)
import functools

import jax
import jax.numpy as jnp
from jax import lax
from jax.experimental import pallas as pl
from jax.experimental.pallas import tpu as pltpu
from jax.experimental.pallas import tpu_sc as plsc

D_MODEL_ = 1024
CHUNK_ROWS = 16
CHUNK = CHUNK_ROWS * D_MODEL_
NBUF = 2


def _sc_add_kernel(x_hbm, t_hbm, o_hbm, xbuf, tbuf, semx, semt, semo,
                   *, rows_per_worker, seq_len, n_workers):
    wid = lax.axis_index("s") * 2 + lax.axis_index("c")
    row0 = wid * rows_per_worker
    elem0 = row0 * D_MODEL_
    telem0 = (row0 % seq_len) * D_MODEL_
    n_chunks = rows_per_worker // CHUNK_ROWS

    def start_in(g, slot):
        cx = pltpu.async_copy(
            x_hbm.at[pl.ds(elem0 + g * CHUNK, CHUNK)], xbuf.at[slot], semx.at[slot])
        ct = pltpu.async_copy(
            t_hbm.at[pl.ds(telem0 + g * CHUNK, CHUNK)], tbuf.at[slot], semt.at[slot])
        return cx, ct

    def start_out(g, slot):
        return pltpu.async_copy(
            xbuf.at[slot], o_hbm.at[pl.ds(elem0 + g * CHUNK, CHUNK)], semo.at[slot])

    pending_in = [start_in(g, g % NBUF) for g in range(NBUF)]
    pending_out = [None] * NBUF

    for g in range(n_chunks):
        slot = g % NBUF
        cx, ct = pending_in[slot]
        cx.wait()
        ct.wait()

        def add_body(i, _):
            off = i * 128
            for u in range(8):
                s = off + u * 16
                plsc.addupdate(xbuf.at[slot, pl.ds(s, 16)], tbuf[slot, pl.ds(s, 16)])
            return 0

        lax.fori_loop(0, CHUNK // 128, add_body, 0)

        pending_out[slot] = start_out(g, slot)
        if g + NBUF < n_chunks:
            pending_out[slot].wait()
            pending_in[slot] = start_in(g + NBUF, slot)

    for slot in range(NBUF):
        if pending_out[slot] is not None:
            pending_out[slot].wait()


def kernel(x, table):
    batch, seq_len, d_model = x.shape
    n_workers = 32
    rows_per_worker = (batch * seq_len) // n_workers

    mesh = plsc.VectorSubcoreMesh(core_axis_name="c", subcore_axis_name="s")
    sc_call = pl.kernel(
        functools.partial(
            _sc_add_kernel,
            rows_per_worker=rows_per_worker,
            seq_len=seq_len,
            n_workers=n_workers,
        ),
        mesh=mesh,
        out_type=jax.ShapeDtypeStruct((batch * seq_len * d_model,), jnp.float32),
        scratch_types=[
            pltpu.VMEM((NBUF, CHUNK), jnp.float32),
            pltpu.VMEM((NBUF, CHUNK), jnp.float32),
            pltpu.SemaphoreType.DMA((NBUF,)),
            pltpu.SemaphoreType.DMA((NBUF,)),
            pltpu.SemaphoreType.DMA((NBUF,)),
        ],
    )
    out = sc_call(x.reshape(-1), table[:seq_len].reshape(-1))
    return out.reshape(batch, seq_len, d_model)
